# baseline (speedup 1.0000x reference)
import functools

import jax
import jax.numpy as jnp
from jax.experimental import pallas as pl
from jax.experimental.pallas import tpu as pltpu

N_EXP_ = 64
TOP_K_ = 8
CAP_FACTOR_ = 1.25


def _shift_down(s, sh):
    pad = jnp.zeros((sh, s.shape[1]), s.dtype)
    return jnp.concatenate([pad, s[:-sh]], axis=0)


def _topk_body(x_ref, w_ref, idx_ref, probs_ref, hist_ref, *, k, n_exp):
    blk = pl.program_id(0)
    logits = jax.lax.dot_general(
        x_ref[...], w_ref[...], (((1,), (1,)), ((), ())),
        preferred_element_type=jnp.float32)
    iota_e = jax.lax.broadcasted_iota(jnp.int32, logits.shape, 1)
    neg_inf = jnp.float32(-jnp.inf)
    vals, idxs, hist_rows = [], [], []
    l = logits
    for _ in range(k):
        m = jnp.max(l, axis=1, keepdims=True)
        a = jnp.min(jnp.where(l == m, iota_e, n_exp), axis=1, keepdims=True)
        sel = iota_e == a
        vals.append(m)
        idxs.append(a)
        hist_rows.append(jnp.sum(sel.astype(jnp.int32), axis=0, keepdims=True))
        l = jnp.where(sel, neg_inf, l)
    v = jnp.concatenate(vals, axis=1)
    p = jnp.exp(v - v[:, 0:1])
    p = p / jnp.sum(p, axis=1, keepdims=True)
    idx_ref[...] = jnp.concatenate(idxs, axis=1)
    probs_ref[...] = p
    h = jnp.concatenate(hist_rows, axis=0)

    @pl.when(blk == 0)
    def _():
        hist_ref[...] = h

    @pl.when(blk != 0)
    def _():
        hist_ref[...] = hist_ref[...] + h


def _rank_body(tot_ref, idx_ref, probs_ref, mask_ref, rank_ref, pm_ref,
               cnt_ref, *, k, n_exp, bm, cap):
    blk = pl.program_id(0)

    @pl.when(blk == 0)
    def _():
        tot = tot_ref[...]
        s = tot
        sh = 1
        while sh < k:
            s = s + _shift_down(s, sh)
            sh *= 2
        cnt_ref[...] = s - tot

    idx = idx_ref[...]
    cnt = cnt_ref[...]
    iota_e = jax.lax.broadcasted_iota(jnp.int32, (bm, n_exp), 1)
    mask_sl, rank_cols, pm_cols, cnt_rows = [], [], [], []
    for j in range(k):
        a = idx[:, j:j + 1]
        oh = (iota_e == a).astype(jnp.int32)
        s = oh
        sh = 1
        while sh < bm:
            s = s + _shift_down(s, sh)
            sh *= 2
        pos = s + cnt[j:j + 1, :]
        rank = jnp.sum(pos * oh, axis=1, keepdims=True) - 1
        ok = rank < cap
        mask_sl.append((oh * ok.astype(jnp.int32))[:, None, :])
        rank_cols.append(rank)
        pm_cols.append(probs_ref[:, j:j + 1] * ok.astype(jnp.float32))
        cnt_rows.append(cnt[j:j + 1, :] + s[bm - 1:bm, :])
    cnt_ref[...] = jnp.concatenate(cnt_rows, axis=0)
    mask_ref[...] = jnp.concatenate(mask_sl, axis=1)
    rank_ref[...] = jnp.concatenate(rank_cols, axis=1)
    pm_ref[...] = jnp.concatenate(pm_cols, axis=1)


@functools.partial(jax.jit, static_argnames=())
def kernel(x, w_g):
    b, t, c = x.shape
    n = b * t
    e = w_g.shape[0]
    k = TOP_K_
    cap = max(int(k * CAP_FACTOR_ * n / e), 4)
    x_flat = x.reshape(n, c)
    bm = 512 if n % 512 == 0 else 128
    nb = n // bm

    idx, probs, tot = pl.pallas_call(
        functools.partial(_topk_body, k=k, n_exp=e),
        grid=(nb,),
        in_specs=[
            pl.BlockSpec((bm, c), lambda i: (i, 0)),
            pl.BlockSpec((e, c), lambda i: (0, 0)),
        ],
        out_specs=[
            pl.BlockSpec((bm, k), lambda i: (i, 0)),
            pl.BlockSpec((bm, k), lambda i: (i, 0)),
            pl.BlockSpec((k, e), lambda i: (0, 0)),
        ],
        out_shape=[
            jax.ShapeDtypeStruct((n, k), jnp.int32),
            jax.ShapeDtypeStruct((n, k), jnp.float32),
            jax.ShapeDtypeStruct((k, e), jnp.int32),
        ],
        compiler_params=pltpu.CompilerParams(
            dimension_semantics=("arbitrary",)),
    )(x_flat, w_g)

    mask, rank, pm = pl.pallas_call(
        functools.partial(_rank_body, k=k, n_exp=e, bm=bm, cap=cap),
        grid=(nb,),
        in_specs=[
            pl.BlockSpec((k, e), lambda i: (0, 0)),
            pl.BlockSpec((bm, k), lambda i: (i, 0)),
            pl.BlockSpec((bm, k), lambda i: (i, 0)),
        ],
        out_specs=[
            pl.BlockSpec((bm, k, e), lambda i: (i, 0, 0)),
            pl.BlockSpec((bm, k), lambda i: (i, 0)),
            pl.BlockSpec((bm, k), lambda i: (i, 0)),
        ],
        out_shape=[
            jax.ShapeDtypeStruct((n, k, e), jnp.int32),
            jax.ShapeDtypeStruct((n, k), jnp.int32),
            jax.ShapeDtypeStruct((n, k), jnp.float32),
        ],
        scratch_shapes=[pltpu.VMEM((k, e), jnp.int32)],
        compiler_params=pltpu.CompilerParams(
            dimension_semantics=("arbitrary",)),
    )(tot, idx, probs)

    return (mask, pm, idx, rank, cap)


# speedup vs baseline: 1.4094x; 1.4094x over previous
import functools

import jax
import jax.numpy as jnp
from jax.experimental import pallas as pl
from jax.experimental.pallas import tpu as pltpu

N_EXP_ = 64
TOP_K_ = 8
CAP_FACTOR_ = 1.25


def _topk_body(x_ref, w_ref, idx_ref, probs_ref, *, k, n_exp, bm):
    logits = jax.lax.dot_general(
        w_ref[...], x_ref[...], (((1,), (1,)), ((), ())),
        preferred_element_type=jnp.float32)
    iota_sub = jax.lax.broadcasted_iota(jnp.int32, (n_exp, bm), 0)
    neg_inf = jnp.float32(-jnp.inf)
    l = logits
    vals, idxs = [], []
    for _ in range(k):
        m = jnp.max(l, axis=0, keepdims=True)
        a = jnp.min(jnp.where(l == m, iota_sub, n_exp), axis=0,
                    keepdims=True)
        sel = iota_sub == a
        l = jnp.where(sel, neg_inf, l)
        vals.append(m)
        idxs.append(a)
    v = jnp.concatenate(vals, axis=0)
    p = jnp.exp(v - v[0:1, :])
    p = p / jnp.sum(p, axis=0, keepdims=True)
    idx_ref[...] = jnp.concatenate(idxs, axis=0)
    probs_ref[...] = p


def _hist_body(idx_ref, gt_ref, hist_ref, *, k, n_exp, bm):
    blk = pl.program_id(0)
    f = n_exp * k
    idx_bf = idx_ref[...].astype(jnp.bfloat16)
    idxbig = jax.lax.dot_general(
        idx_bf, gt_ref[...], (((1,), (0,)), ((), ())),
        preferred_element_type=jnp.float32)
    jmod = (jax.lax.broadcasted_iota(jnp.int32, (bm, f), 1)
            & (n_exp - 1)).astype(jnp.float32)
    oh = (idxbig == jmod).astype(jnp.int32)
    h = jnp.sum(oh, axis=0, keepdims=True)

    @pl.when(blk == 0)
    def _():
        hist_ref[...] = jnp.zeros_like(hist_ref)

    hist_ref[0:1, :] = hist_ref[0:1, :] + h


def _shift_lanes(s, sh):
    pad = jnp.zeros((s.shape[0], sh), s.dtype)
    return jnp.concatenate([pad, s[:, :-sh]], axis=1)


def _rank_body(hist_ref, idx_ref, probs_ref, l_ref, gt_ref, g_ref,
               mask_ref, rank_ref, pm_ref, cnt_ref,
               *, k, n_exp, bm, cap):
    blk = pl.program_id(0)
    f = n_exp * k

    @pl.when(blk == 0)
    def _():
        tot = hist_ref[0:1, :].astype(jnp.float32)
        s = tot
        sh = n_exp
        while sh < f:
            s = s + _shift_lanes(s, sh)
            sh *= 2
        cnt_ref[...] = s - tot

    idx = idx_ref[...]
    idx_bf = idx.astype(jnp.bfloat16)
    idxbig = jax.lax.dot_general(
        idx_bf, gt_ref[...], (((1,), (0,)), ((), ())),
        preferred_element_type=jnp.float32)
    jmod = (jax.lax.broadcasted_iota(jnp.int32, (bm, f), 1)
            & (n_exp - 1)).astype(jnp.float32)
    ohf = (idxbig == jmod).astype(jnp.float32)
    posl = jax.lax.dot_general(
        l_ref[...], ohf.astype(jnp.bfloat16), (((1,), (0,)), ((), ())),
        preferred_element_type=jnp.float32)
    t1 = ohf * (posl + cnt_ref[0:1, :])
    rank_f = jax.lax.dot_general(
        t1, g_ref[...], (((1,), (0,)), ((), ())),
        preferred_element_type=jnp.float32,
        precision=jax.lax.Precision.HIGHEST)
    rank = rank_f.astype(jnp.int32) - 1
    ok = rank < cap
    cnt_ref[...] = cnt_ref[...] + posl[bm - 1:bm, :]
    rank_ref[...] = rank
    pm_ref[...] = probs_ref[...] * ok.astype(jnp.float32)
    iota_e = jax.lax.broadcasted_iota(jnp.int32, (bm, n_exp), 1)
    for j in range(k):
        ohj = iota_e == idx[:, j:j + 1]
        mask_ref[:, j, :] = (ohj & ok[:, j:j + 1]).astype(jnp.int32)


@functools.partial(jax.jit, static_argnames=())
def kernel(x, w_g):
    b, t, c = x.shape
    n = b * t
    e = w_g.shape[0]
    k = TOP_K_
    f = e * k
    cap = max(int(k * CAP_FACTOR_ * n / e), 4)
    x_flat = x.reshape(n, c)
    bm = 512 if n % 512 == 0 else 128
    nb = n // bm

    idx_t, probs_t = pl.pallas_call(
        functools.partial(_topk_body, k=k, n_exp=e, bm=bm),
        grid=(nb,),
        in_specs=[
            pl.BlockSpec((bm, c), lambda i: (i, 0)),
            pl.BlockSpec((e, c), lambda i: (0, 0)),
        ],
        out_specs=[
            pl.BlockSpec((k, bm), lambda i: (0, i)),
            pl.BlockSpec((k, bm), lambda i: (0, i)),
        ],
        out_shape=[
            jax.ShapeDtypeStruct((k, n), jnp.int32),
            jax.ShapeDtypeStruct((k, n), jnp.float32),
        ],
        compiler_params=pltpu.CompilerParams(
            dimension_semantics=("arbitrary",)),
    )(x_flat, w_g)

    idx = idx_t.T
    probs = probs_t.T

    kk = jnp.arange(k, dtype=jnp.int32)
    jj = jnp.arange(f, dtype=jnp.int32)
    gt_bf = (jj[None, :] // e == kk[:, None]).astype(jnp.bfloat16)
    g_f32 = gt_bf.T.astype(jnp.float32)
    ii = jnp.arange(bm, dtype=jnp.int32)
    l_bf = (ii[:, None] >= ii[None, :]).astype(jnp.bfloat16)

    hist = pl.pallas_call(
        functools.partial(_hist_body, k=k, n_exp=e, bm=bm),
        grid=(nb,),
        in_specs=[
            pl.BlockSpec((bm, k), lambda i: (i, 0)),
            pl.BlockSpec((k, f), lambda i: (0, 0)),
        ],
        out_specs=pl.BlockSpec((8, f), lambda i: (0, 0)),
        out_shape=jax.ShapeDtypeStruct((8, f), jnp.int32),
        compiler_params=pltpu.CompilerParams(
            dimension_semantics=("arbitrary",)),
    )(idx, gt_bf)

    mask, rank, pm = pl.pallas_call(
        functools.partial(_rank_body, k=k, n_exp=e, bm=bm, cap=cap),
        grid=(nb,),
        in_specs=[
            pl.BlockSpec((8, f), lambda i: (0, 0)),
            pl.BlockSpec((bm, k), lambda i: (i, 0)),
            pl.BlockSpec((bm, k), lambda i: (i, 0)),
            pl.BlockSpec((bm, bm), lambda i: (0, 0)),
            pl.BlockSpec((k, f), lambda i: (0, 0)),
            pl.BlockSpec((f, k), lambda i: (0, 0)),
        ],
        out_specs=[
            pl.BlockSpec((bm, k, e), lambda i: (i, 0, 0)),
            pl.BlockSpec((bm, k), lambda i: (i, 0)),
            pl.BlockSpec((bm, k), lambda i: (i, 0)),
        ],
        out_shape=[
            jax.ShapeDtypeStruct((n, k, e), jnp.int32),
            jax.ShapeDtypeStruct((n, k), jnp.int32),
            jax.ShapeDtypeStruct((n, k), jnp.float32),
        ],
        scratch_shapes=[pltpu.VMEM((1, f), jnp.float32)],
        compiler_params=pltpu.CompilerParams(
            dimension_semantics=("arbitrary",)),
    )(hist, idx, probs, l_bf, gt_bf, g_f32)

    return (mask, pm, idx, rank, cap)
